# baseline (speedup 1.0000x reference)
import jax
import jax.numpy as jnp
from jax import lax
from jax.experimental import pallas as pl
from jax.experimental.pallas import tpu as pltpu
from jax.experimental.pallas import tpu_sc as plsc

VOCAB = 1_000_000
EMB = 64
B_TOTAL = 4096 * 200

_info = plsc.get_sparse_core_info()
NC, NS = _info.num_cores, _info.num_subcores
NW = NC * NS

CHUNK = 128
PER_W = B_TOTAL // NW
N_CHUNKS = PER_W // CHUNK


def _body(idx_hbm, table_hbm, out_hbm, idx_v, buf, gsem):
    wid = lax.axis_index("s") * NC + lax.axis_index("c")
    pltpu.sync_copy(idx_hbm.at[wid], idx_v)
    base = wid * PER_W

    @pl.loop(0, N_CHUNKS)
    def chunk_loop(j):
        pltpu.async_copy(table_hbm.at[idx_v.at[j]], buf, gsem).wait()
        pltpu.sync_copy(buf, out_hbm.at[pl.ds(base + j * CHUNK, CHUNK)])


@jax.jit
def _embed(idx_grouped, table):
    mesh = plsc.VectorSubcoreMesh(core_axis_name="c", subcore_axis_name="s")
    k = pl.kernel(
        _body,
        out_type=jax.ShapeDtypeStruct((B_TOTAL, EMB), jnp.float32),
        mesh=mesh,
        scratch_types=[
            pltpu.VMEM((N_CHUNKS, CHUNK), jnp.int32),
            pltpu.VMEM((CHUNK, EMB), jnp.float32),
            pltpu.SemaphoreType.DMA,
        ],
        compiler_params=pltpu.CompilerParams(use_tc_tiling_on_sc=False),
    )
    return k(idx_grouped, table)


def kernel(input_vars, table):
    idx = input_vars.reshape(NW, N_CHUNKS, CHUNK).astype(jnp.int32)
    out = _embed(idx, table)
    return out.reshape(input_vars.shape[0], input_vars.shape[1], EMB)


# speedup vs baseline: 1.1178x; 1.1178x over previous
import jax
import jax.numpy as jnp
from jax import lax
from jax.experimental import pallas as pl
from jax.experimental.pallas import tpu as pltpu
from jax.experimental.pallas import tpu_sc as plsc

VOCAB = 1_000_000
EMB = 64
B_TOTAL = 4096 * 200

_info = plsc.get_sparse_core_info()
NC, NS = _info.num_cores, _info.num_subcores
NW = NC * NS

CHUNK = 128
PER_W = B_TOTAL // NW
N_CHUNKS = PER_W // CHUNK
G_CHUNKS = 4
G_ROWS = G_CHUNKS * CHUNK
N_GROUPS = N_CHUNKS // G_CHUNKS
N_PAIRS = N_GROUPS // 2


def _body(idx_hbm, table_hbm, out_hbm, idx_v, buf0, buf1, gs0, gs1, ws0, ws1):
    wid = lax.axis_index("s") * NC + lax.axis_index("c")
    pltpu.sync_copy(idx_hbm.at[wid], idx_v)
    base = wid * PER_W

    bufs = (buf0, buf1)
    gsems = (gs0, gs1)
    wsems = (ws0, ws1)

    def issue_gathers(g, buf, gsem):
        for c in range(G_CHUNKS):
            pltpu.async_copy(
                table_hbm.at[idx_v.at[g * G_CHUNKS + c]],
                buf.at[pl.ds(c * CHUNK, CHUNK)],
                gsem,
            )

    def drain_gathers(buf, gsem):
        pltpu.make_async_copy(table_hbm.at[pl.ds(0, G_ROWS)], buf, gsem).wait()

    def wait_write(g, buf, wsem):
        pltpu.make_async_copy(
            buf, out_hbm.at[pl.ds(base + g * G_ROWS, G_ROWS)], wsem
        ).wait()

    issue_gathers(0, buf0, gs0)

    @pl.loop(0, N_PAIRS)
    def pair_loop(t):
        for b in range(2):
            g = 2 * t + b
            nxt = g + 1
            ob = 1 - b

            @pl.when(nxt < N_GROUPS)
            def _issue_next():
                @pl.when(g >= 1)
                def _wait_prev_write():
                    wait_write(g - 1, bufs[ob], wsems[ob])

                issue_gathers(nxt, bufs[ob], gsems[ob])

            drain_gathers(bufs[b], gsems[b])
            pltpu.async_copy(
                bufs[b], out_hbm.at[pl.ds(base + g * G_ROWS, G_ROWS)], wsems[b]
            )

    wait_write(N_GROUPS - 2, bufs[(N_GROUPS - 2) % 2], wsems[(N_GROUPS - 2) % 2])
    wait_write(N_GROUPS - 1, bufs[(N_GROUPS - 1) % 2], wsems[(N_GROUPS - 1) % 2])


@jax.jit
def _embed(idx_grouped, table):
    mesh = plsc.VectorSubcoreMesh(core_axis_name="c", subcore_axis_name="s")
    k = pl.kernel(
        _body,
        out_type=jax.ShapeDtypeStruct((B_TOTAL, EMB), jnp.float32),
        mesh=mesh,
        scratch_types=[
            pltpu.VMEM((N_CHUNKS, CHUNK), jnp.int32),
            pltpu.VMEM((G_ROWS, EMB), jnp.float32),
            pltpu.VMEM((G_ROWS, EMB), jnp.float32),
            pltpu.SemaphoreType.DMA,
            pltpu.SemaphoreType.DMA,
            pltpu.SemaphoreType.DMA,
            pltpu.SemaphoreType.DMA,
        ],
        compiler_params=pltpu.CompilerParams(use_tc_tiling_on_sc=False),
    )
    return k(idx_grouped, table)


def kernel(input_vars, table):
    idx = input_vars.reshape(NW, N_CHUNKS, CHUNK).astype(jnp.int32)
    out = _embed(idx, table)
    return out.reshape(input_vars.shape[0], input_vars.shape[1], EMB)


# speedup vs baseline: 1.1181x; 1.0003x over previous
import jax
import jax.numpy as jnp
from jax import lax
from jax.experimental import pallas as pl
from jax.experimental.pallas import tpu as pltpu
from jax.experimental.pallas import tpu_sc as plsc

VOCAB = 1_000_000
EMB = 64
B_TOTAL = 4096 * 200

_info = plsc.get_sparse_core_info()
NC, NS = _info.num_cores, _info.num_subcores
NW = NC * NS

CHUNK = 128
PER_W = B_TOTAL // NW
N_CHUNKS = PER_W // CHUNK
NBUF = 8
AHEAD = NBUF - 1


def _body(idx_hbm, table_hbm, out_hbm, idx_v, buf, gsem, wsem):
    wid = lax.axis_index("s") * NC + lax.axis_index("c")
    pltpu.sync_copy(idx_hbm.at[wid], idx_v)
    base = wid * PER_W

    def issue_gather(j, b):
        pltpu.async_copy(table_hbm.at[idx_v.at[j]], buf.at[b], gsem.at[b])

    def drain_gather(b):
        pltpu.make_async_copy(
            table_hbm.at[pl.ds(0, CHUNK)], buf.at[b], gsem.at[b]
        ).wait()

    def issue_write(j, b):
        pltpu.async_copy(
            buf.at[b], out_hbm.at[pl.ds(base + j * CHUNK, CHUNK)], wsem.at[b]
        )

    def wait_write(j, b):
        pltpu.make_async_copy(
            buf.at[b], out_hbm.at[pl.ds(base + j * CHUNK, CHUNK)], wsem.at[b]
        ).wait()

    for j in range(AHEAD):
        issue_gather(j, j)

    @pl.loop(0, N_CHUNKS)
    def chunk_loop(j):
        b = lax.rem(j, NBUF)
        nxt = j + AHEAD

        @pl.when(nxt < N_CHUNKS)
        def _issue_next():
            nb = lax.rem(nxt, NBUF)

            @pl.when(j >= 1)
            def _wait_prev_write():
                wait_write(j - 1, nb)

            issue_gather(nxt, nb)

        drain_gather(b)
        issue_write(j, b)

    @pl.loop(N_CHUNKS - AHEAD, N_CHUNKS)
    def tail_loop(j):
        wait_write(j, lax.rem(j, NBUF))


@jax.jit
def _embed(idx_grouped, table):
    mesh = plsc.VectorSubcoreMesh(core_axis_name="c", subcore_axis_name="s")
    k = pl.kernel(
        _body,
        out_type=jax.ShapeDtypeStruct((B_TOTAL, EMB), jnp.float32),
        mesh=mesh,
        scratch_types=[
            pltpu.VMEM((N_CHUNKS, CHUNK), jnp.int32),
            pltpu.VMEM((NBUF, CHUNK, EMB), jnp.float32),
            pltpu.SemaphoreType.DMA((NBUF,)),
            pltpu.SemaphoreType.DMA((NBUF,)),
        ],
        compiler_params=pltpu.CompilerParams(use_tc_tiling_on_sc=False),
    )
    return k(idx_grouped, table)


def kernel(input_vars, table):
    idx = input_vars.reshape(NW, N_CHUNKS, CHUNK).astype(jnp.int32)
    out = _embed(idx, table)
    return out.reshape(input_vars.shape[0], input_vars.shape[1], EMB)


# speedup vs baseline: 1.1187x; 1.0005x over previous
import jax
import jax.numpy as jnp
from jax import lax
from jax.experimental import pallas as pl
from jax.experimental.pallas import tpu as pltpu
from jax.experimental.pallas import tpu_sc as plsc

VOCAB = 1_000_000
EMB = 64
N_SEQ = 4096
SEQ_LEN = 200

_info = plsc.get_sparse_core_info()
NC, NS = _info.num_cores, _info.num_subcores
NW = NC * NS
SEQ_PER_W = N_SEQ // NW
SPLITS = ((0, 128), (128, 72))
NBUF = 4
AHEAD = NBUF - 1


def _body(idx_hbm, table_hbm, out_hbm, idx_v, buf, gsem, wsem):
    wid = lax.axis_index("s") * NC + lax.axis_index("c")
    seq0 = wid * SEQ_PER_W
    pltpu.sync_copy(idx_hbm.at[pl.ds(seq0, SEQ_PER_W)], idx_v)

    def issue_gathers(s, b):
        for off, size in SPLITS:
            pltpu.async_copy(
                table_hbm.at[idx_v.at[s, pl.ds(off, size)]],
                buf.at[b, pl.ds(off, size)],
                gsem.at[b],
            )

    def drain_gathers(b):
        pltpu.make_async_copy(
            table_hbm.at[pl.ds(0, SEQ_LEN)], buf.at[b], gsem.at[b]
        ).wait()

    def issue_write(s, b):
        pltpu.async_copy(buf.at[b], out_hbm.at[seq0 + s], wsem.at[b])

    def wait_write(s, b):
        pltpu.make_async_copy(
            buf.at[b], out_hbm.at[seq0 + s], wsem.at[b]
        ).wait()

    for s in range(AHEAD):
        issue_gathers(s, s)

    @pl.loop(0, SEQ_PER_W)
    def seq_loop(s):
        b = lax.rem(s, NBUF)
        nxt = s + AHEAD

        @pl.when(nxt < SEQ_PER_W)
        def _issue_next():
            nb = lax.rem(nxt, NBUF)

            @pl.when(s >= 1)
            def _wait_prev_write():
                wait_write(s - 1, nb)

            issue_gathers(nxt, nb)

        drain_gathers(b)
        issue_write(s, b)

    @pl.loop(SEQ_PER_W - AHEAD, SEQ_PER_W)
    def tail_loop(s):
        wait_write(s, lax.rem(s, NBUF))


@jax.jit
def _embed(idx, table):
    mesh = plsc.VectorSubcoreMesh(core_axis_name="c", subcore_axis_name="s")
    k = pl.kernel(
        _body,
        out_type=jax.ShapeDtypeStruct((N_SEQ, SEQ_LEN, EMB), jnp.float32),
        mesh=mesh,
        scratch_types=[
            pltpu.VMEM((SEQ_PER_W, SEQ_LEN), jnp.int32),
            pltpu.VMEM((NBUF, SEQ_LEN, EMB), jnp.float32),
            pltpu.SemaphoreType.DMA((NBUF,)),
            pltpu.SemaphoreType.DMA((NBUF,)),
        ],
        compiler_params=pltpu.CompilerParams(use_tc_tiling_on_sc=False),
    )
    return k(idx, table)


def kernel(input_vars, table):
    return _embed(input_vars.astype(jnp.int32), table)
